# Initial kernel scaffold; baseline (speedup 1.0000x reference)
import functools

import jax
import jax.numpy as jnp
from jax import lax
from jax.experimental import pallas as pl
from jax.experimental.pallas import tpu as pltpu
from jax.experimental.pallas import tpu_sc as plsc

B = 16384
DIM = 128
_ROWS = 7 * 12 * 31


def _fuse_tables(week_emb, month_emb, day_emb):
    def body(w_ref, m_ref, d_ref, o_ref):
        o_ref[...] = (w_ref[...][:, None, None, :]
                      + m_ref[...][None, :, None, :]
                      + d_ref[...][None, None, :, :])

    return pl.pallas_call(
        body,
        out_shape=jax.ShapeDtypeStruct((7, 12, 31, DIM), jnp.float32),
    )(week_emb, month_emb, day_emb)


def _date_indices(tt):
    days = lax.div(tt, 86400)
    week = lax.rem(days + 3, 7)
    z = days + 719468
    era = lax.div(z, 146097)
    doe = z - era * 146097
    yoe = lax.div(
        doe - lax.div(doe, 1460) + lax.div(doe, 36524) - lax.div(doe, 146096),
        365)
    doy = doe - (365 * yoe + lax.div(yoe, 4) - lax.div(yoe, 100))
    mp = lax.div(5 * doy + 2, 153)
    d0 = doy - lax.div(153 * mp + 2, 5)
    m0 = mp + jnp.where(mp < 10, 2, -10)
    return week * 372 + m0 * 31 + d0


def _sc_lookup(t, fused):
    info = plsc.get_sparse_core_info()
    nc, ns = info.num_cores, info.num_subcores
    nw = nc * ns
    bpw = B // nw
    n_chunk = bpw // 128
    mesh = plsc.VectorSubcoreMesh(core_axis_name="c", subcore_axis_name="s")

    @functools.partial(
        pl.kernel, mesh=mesh,
        out_type=jax.ShapeDtypeStruct((B, DIM), jnp.float32),
        scratch_types=[
            pltpu.VMEM((bpw,), jnp.int32),
            pltpu.VMEM((n_chunk, 128), jnp.int32),
            pltpu.VMEM((bpw, DIM), jnp.float32),
            pltpu.SemaphoreType.DMA,
        ],
    )
    def k(t_hbm, fused_hbm, out_hbm, t_v, idx_v, rows_v, sem):
        wid = lax.axis_index("s") * nc + lax.axis_index("c")
        base = wid * bpw
        pltpu.sync_copy(t_hbm.at[pl.ds(base, bpw)], t_v)
        copies = []
        for r in range(n_chunk):
            def step(c, carry, r=r):
                tt = t_v[pl.ds(r * 128 + c * 16, 16)]
                idx_v[r, pl.ds(c * 16, 16)] = _date_indices(tt)
                return carry
            lax.fori_loop(0, 128 // 16, step, 0)
            copies.append(pltpu.async_copy(
                fused_hbm.at[idx_v.at[r]],
                rows_v.at[pl.ds(r * 128, 128)], sem))
        for cp in copies:
            cp.wait()
        pltpu.sync_copy(rows_v, out_hbm.at[pl.ds(base, bpw)])

    return k(t, fused)


def kernel(t, week_emb, month_emb, day_emb):
    fused = _fuse_tables(week_emb, month_emb, day_emb).reshape(_ROWS, DIM)
    return _sc_lookup(t.astype(jnp.int32), fused)


# speedup vs baseline: 5.9513x; 5.9513x over previous
import functools

import jax
import jax.numpy as jnp
from jax import lax
from jax.experimental import pallas as pl
from jax.experimental.pallas import tpu as pltpu
from jax.experimental.pallas import tpu_sc as plsc

B = 16384
DIM = 128
_ROWS = 7 * 12 * 31


def _fuse_tables(week_emb, month_emb, day_emb):
    def body(w_ref, m_ref, d_ref, o_ref):
        o_ref[...] = (w_ref[...][:, None, None, :]
                      + m_ref[...][None, :, None, :]
                      + d_ref[...][None, None, :, :])

    return pl.pallas_call(
        body,
        out_shape=jax.ShapeDtypeStruct((7, 12, 31, DIM), jnp.float32),
    )(week_emb, month_emb, day_emb)


def _fdiv(xf, c):
    return ((xf + 0.5) * (1.0 / c)).astype(jnp.int32)


def _date_indices(tt):
    tf = tt.astype(jnp.float32)
    q = (tf * (1.0 / 86400.0)).astype(jnp.int32)
    r = tt - q * 86400
    q = jnp.where(r >= 86400, q + 1, q)
    q = jnp.where(r < 0, q - 1, q)
    days_f = q.astype(jnp.float32)
    week = (q + 3) - 7 * _fdiv(days_f + 3.0, 7)
    zf = days_f + 719468.0
    era = _fdiv(zf, 146097)
    doe_f = zf - era.astype(jnp.float32) * 146097.0
    yoe = _fdiv(doe_f - _fdiv(doe_f, 1460).astype(jnp.float32)
                + _fdiv(doe_f, 36524).astype(jnp.float32)
                - _fdiv(doe_f, 146096).astype(jnp.float32), 365)
    yoe_f = yoe.astype(jnp.float32)
    doy_f = doe_f - (365.0 * yoe_f + _fdiv(yoe_f, 4).astype(jnp.float32)
                     - _fdiv(yoe_f, 100).astype(jnp.float32))
    mp = _fdiv(5.0 * doy_f + 2.0, 153)
    d0 = doy_f.astype(jnp.int32) - _fdiv(153.0 * mp.astype(jnp.float32) + 2.0, 5)
    m0 = mp + jnp.where(mp < 10, 2, -10)
    return week * 372 + m0 * 31 + d0


def _sc_lookup(t, fused):
    info = plsc.get_sparse_core_info()
    nc, ns = info.num_cores, info.num_subcores
    nw = nc * ns
    bpw = B // nw
    n_chunk = bpw // 128
    mesh = plsc.VectorSubcoreMesh(core_axis_name="c", subcore_axis_name="s")

    @functools.partial(
        pl.kernel, mesh=mesh,
        out_type=jax.ShapeDtypeStruct((B, DIM), jnp.float32),
        scratch_types=[
            pltpu.VMEM((bpw,), jnp.int32),
            pltpu.VMEM((n_chunk, 128), jnp.int32),
            pltpu.VMEM((bpw, DIM), jnp.float32),
            pltpu.SemaphoreType.DMA,
        ],
    )
    def k(t_hbm, fused_hbm, out_hbm, t_v, idx_v, rows_v, sem):
        wid = lax.axis_index("s") * nc + lax.axis_index("c")
        base = wid * bpw
        pltpu.sync_copy(t_hbm.at[pl.ds(base, bpw)], t_v)
        copies = []
        for r in range(n_chunk):
            def step(c, carry, r=r):
                tt = t_v[pl.ds(r * 128 + c * 16, 16)]
                idx_v[r, pl.ds(c * 16, 16)] = _date_indices(tt)
                return carry
            lax.fori_loop(0, 128 // 16, step, 0)
            copies.append(pltpu.async_copy(
                fused_hbm.at[idx_v.at[r]],
                rows_v.at[pl.ds(r * 128, 128)], sem))
        for cp in copies:
            cp.wait()
        pltpu.sync_copy(rows_v, out_hbm.at[pl.ds(base, bpw)])

    return k(t, fused)


def kernel(t, week_emb, month_emb, day_emb):
    fused = _fuse_tables(week_emb, month_emb, day_emb).reshape(_ROWS, DIM)
    return _sc_lookup(t.astype(jnp.int32), fused)


# speedup vs baseline: 5.9569x; 1.0010x over previous
import functools

import jax
import jax.numpy as jnp
import numpy as np
from jax import lax
from jax.experimental import pallas as pl
from jax.experimental.pallas import tpu as pltpu
from jax.experimental.pallas import tpu_sc as plsc

B = 16384
DIM = 128
_ROWS = 7 * 12 * 31


def _three_hot() -> np.ndarray:
    r = np.arange(_ROWS)
    h = np.zeros((_ROWS, 50), np.float32)
    h[r, r // 372] = 1.0
    h[r, 7 + (r // 31) % 12] = 1.0
    h[r, 19 + r % 31] = 1.0
    return h


_H = _three_hot()


def _fuse_tables(week_emb, month_emb, day_emb):
    def body(h_ref, w_ref, m_ref, d_ref, o_ref):
        t = jnp.concatenate([w_ref[...], m_ref[...], d_ref[...]], axis=0)
        o_ref[...] = jnp.dot(h_ref[...], t,
                             preferred_element_type=jnp.float32,
                             precision=lax.Precision.HIGHEST)

    return pl.pallas_call(
        body,
        out_shape=jax.ShapeDtypeStruct((_ROWS, DIM), jnp.float32),
    )(_H, week_emb, month_emb, day_emb)


def _fdiv(xf, c):
    return ((xf + 0.5) * (1.0 / c)).astype(jnp.int32)


def _date_indices(tt):
    tf = tt.astype(jnp.float32)
    q = (tf * (1.0 / 86400.0)).astype(jnp.int32)
    r = tt - q * 86400
    q = jnp.where(r >= 86400, q + 1, q)
    q = jnp.where(r < 0, q - 1, q)
    days_f = q.astype(jnp.float32)
    week = (q + 3) - 7 * _fdiv(days_f + 3.0, 7)
    zf = days_f + 719468.0
    era = _fdiv(zf, 146097)
    doe_f = zf - era.astype(jnp.float32) * 146097.0
    yoe = _fdiv(doe_f - _fdiv(doe_f, 1460).astype(jnp.float32)
                + _fdiv(doe_f, 36524).astype(jnp.float32)
                - _fdiv(doe_f, 146096).astype(jnp.float32), 365)
    yoe_f = yoe.astype(jnp.float32)
    doy_f = doe_f - (365.0 * yoe_f + _fdiv(yoe_f, 4).astype(jnp.float32)
                     - _fdiv(yoe_f, 100).astype(jnp.float32))
    mp = _fdiv(5.0 * doy_f + 2.0, 153)
    d0 = doy_f.astype(jnp.int32) - _fdiv(153.0 * mp.astype(jnp.float32) + 2.0, 5)
    m0 = mp + jnp.where(mp < 10, 2, -10)
    return week * 372 + m0 * 31 + d0


def _sc_lookup(t, fused):
    info = plsc.get_sparse_core_info()
    nc, ns = info.num_cores, info.num_subcores
    nw = nc * ns
    bpw = B // nw
    n_chunk = bpw // 128
    mesh = plsc.VectorSubcoreMesh(core_axis_name="c", subcore_axis_name="s")

    @functools.partial(
        pl.kernel, mesh=mesh,
        out_type=jax.ShapeDtypeStruct((B, DIM), jnp.float32),
        scratch_types=[
            pltpu.VMEM((bpw,), jnp.int32),
            pltpu.VMEM((n_chunk, 128), jnp.int32),
            pltpu.VMEM((bpw, DIM), jnp.float32),
            pltpu.SemaphoreType.DMA,
            pltpu.SemaphoreType.DMA,
        ],
    )
    def k(t_hbm, fused_hbm, out_hbm, t_v, idx_v, rows_v, sem_g, sem_w):
        wid = lax.axis_index("s") * nc + lax.axis_index("c")
        base = wid * bpw
        pltpu.sync_copy(t_hbm.at[pl.ds(base, bpw)], t_v)
        gathers = []
        for r in range(n_chunk):
            def step(c, carry, r=r):
                tt = t_v[pl.ds(r * 128 + c * 16, 16)]
                idx_v[r, pl.ds(c * 16, 16)] = _date_indices(tt)
                return carry
            lax.fori_loop(0, 128 // 16, step, 0)
            gathers.append(pltpu.async_copy(
                fused_hbm.at[idx_v.at[r]],
                rows_v.at[pl.ds(r * 128, 128)], sem_g))
        writes = []
        for r in range(n_chunk):
            gathers[r].wait()
            writes.append(pltpu.async_copy(
                rows_v.at[pl.ds(r * 128, 128)],
                out_hbm.at[pl.ds(base + r * 128, 128)], sem_w))
        for wr in writes:
            wr.wait()

    return k(t, fused)


def kernel(t, week_emb, month_emb, day_emb):
    fused = _fuse_tables(week_emb, month_emb, day_emb).reshape(_ROWS, DIM)
    return _sc_lookup(t.astype(jnp.int32), fused)


# speedup vs baseline: 6.0276x; 1.0119x over previous
import functools

import jax
import jax.numpy as jnp
import numpy as np
from jax import lax
from jax.experimental import pallas as pl
from jax.experimental.pallas import tpu as pltpu
from jax.experimental.pallas import tpu_sc as plsc

B = 16384
DIM = 128
_ROWS = 7 * 12 * 31


def _three_hot() -> np.ndarray:
    r = np.arange(_ROWS)
    h = np.zeros((_ROWS, 50), np.float32)
    h[r, r // 372] = 1.0
    h[r, 7 + (r // 31) % 12] = 1.0
    h[r, 19 + r % 31] = 1.0
    return h


_H = _three_hot()


def _fuse_tables(week_emb, month_emb, day_emb):
    def body(h_ref, w_ref, m_ref, d_ref, o_ref):
        t = jnp.concatenate([w_ref[...], m_ref[...], d_ref[...]], axis=0)
        o_ref[...] = jnp.dot(h_ref[...], t,
                             preferred_element_type=jnp.float32)

    return pl.pallas_call(
        body,
        out_shape=jax.ShapeDtypeStruct((_ROWS, DIM), jnp.float32),
    )(_H, week_emb, month_emb, day_emb)


def _fdiv(xf, c):
    return ((xf + 0.5) * (1.0 / c)).astype(jnp.int32)


def _date_indices(tt):
    tf = tt.astype(jnp.float32)
    q = (tf * (1.0 / 86400.0)).astype(jnp.int32)
    r = tt - q * 86400
    q = jnp.where(r >= 86400, q + 1, q)
    q = jnp.where(r < 0, q - 1, q)
    days_f = q.astype(jnp.float32)
    week = (q + 3) - 7 * _fdiv(days_f + 3.0, 7)
    zf = days_f + 719468.0
    era = _fdiv(zf, 146097)
    doe_f = zf - era.astype(jnp.float32) * 146097.0
    yoe = _fdiv(doe_f - _fdiv(doe_f, 1460).astype(jnp.float32)
                + _fdiv(doe_f, 36524).astype(jnp.float32)
                - _fdiv(doe_f, 146096).astype(jnp.float32), 365)
    yoe_f = yoe.astype(jnp.float32)
    doy_f = doe_f - (365.0 * yoe_f + _fdiv(yoe_f, 4).astype(jnp.float32)
                     - _fdiv(yoe_f, 100).astype(jnp.float32))
    mp = _fdiv(5.0 * doy_f + 2.0, 153)
    d0 = doy_f.astype(jnp.int32) - _fdiv(153.0 * mp.astype(jnp.float32) + 2.0, 5)
    m0 = mp + jnp.where(mp < 10, 2, -10)
    return week * 372 + m0 * 31 + d0


def _sc_lookup(t, fused):
    info = plsc.get_sparse_core_info()
    nc, ns = info.num_cores, info.num_subcores
    nw = nc * ns
    bpw = B // nw
    n_chunk = bpw // 128
    mesh = plsc.VectorSubcoreMesh(core_axis_name="c", subcore_axis_name="s")

    @functools.partial(
        pl.kernel, mesh=mesh,
        out_type=jax.ShapeDtypeStruct((B, DIM), jnp.float32),
        scratch_types=[
            pltpu.VMEM((bpw,), jnp.int32),
            pltpu.VMEM((n_chunk, 128), jnp.int32),
            pltpu.VMEM((bpw, DIM), jnp.float32),
            pltpu.SemaphoreType.DMA,
            pltpu.SemaphoreType.DMA,
        ],
    )
    def k(t_hbm, fused_hbm, out_hbm, t_v, idx_v, rows_v, sem_g, sem_w):
        wid = lax.axis_index("s") * nc + lax.axis_index("c")
        base = wid * bpw
        pltpu.sync_copy(t_hbm.at[pl.ds(base, bpw)], t_v)
        gathers = []
        for r in range(n_chunk):
            def step(c, carry, r=r):
                tt = t_v[pl.ds(r * 128 + c * 16, 16)]
                idx_v[r, pl.ds(c * 16, 16)] = _date_indices(tt)
                return carry
            lax.fori_loop(0, 128 // 16, step, 0)
            gathers.append(pltpu.async_copy(
                fused_hbm.at[idx_v.at[r]],
                rows_v.at[pl.ds(r * 128, 128)], sem_g))
        half = (n_chunk // 2) * 128
        gathers[0].wait()
        gathers[1].wait()
        w0 = pltpu.async_copy(rows_v.at[pl.ds(0, half)],
                              out_hbm.at[pl.ds(base, half)], sem_w)
        gathers[2].wait()
        gathers[3].wait()
        w1 = pltpu.async_copy(rows_v.at[pl.ds(half, half)],
                              out_hbm.at[pl.ds(base + half, half)], sem_w)
        w0.wait()
        w1.wait()

    return k(t, fused)


def kernel(t, week_emb, month_emb, day_emb):
    fused = _fuse_tables(week_emb, month_emb, day_emb).reshape(_ROWS, DIM)
    return _sc_lookup(t.astype(jnp.int32), fused)


# speedup vs baseline: 6.6555x; 1.1042x over previous
import functools

import jax
import jax.numpy as jnp
import numpy as np
from jax import lax
from jax.experimental import pallas as pl
from jax.experimental.pallas import tpu as pltpu
from jax.experimental.pallas import tpu_sc as plsc

B = 16384
DIM = 128
_ROWS = 7 * 12 * 31
_ROWS_PAD = 2688


def _three_hot() -> np.ndarray:
    r = np.arange(_ROWS)
    h = np.zeros((_ROWS_PAD, 50), np.float32)
    h[r, r // 372] = 1.0
    h[r, 7 + (r // 31) % 12] = 1.0
    h[r, 19 + r % 31] = 1.0
    return h


_H = _three_hot()


def _fuse_tables(week_emb, month_emb, day_emb):
    def body(h_ref, w_ref, m_ref, d_ref, o_ref):
        t = jnp.concatenate([w_ref[...], m_ref[...], d_ref[...]], axis=0)
        o_ref[...] = jnp.dot(h_ref[...], t,
                             preferred_element_type=jnp.float32)

    return pl.pallas_call(
        body,
        out_shape=jax.ShapeDtypeStruct((_ROWS_PAD, DIM), jnp.float32),
    )(_H, week_emb, month_emb, day_emb)


def _fdiv(xf, c):
    return ((xf + 0.5) * (1.0 / c)).astype(jnp.int32)


def _date_indices(tt):
    tf = tt.astype(jnp.float32)
    q = (tf * (1.0 / 86400.0)).astype(jnp.int32)
    r = tt - q * 86400
    q = jnp.where(r >= 86400, q + 1, q)
    q = jnp.where(r < 0, q - 1, q)
    days_f = q.astype(jnp.float32)
    week = (q + 3) - 7 * _fdiv(days_f + 3.0, 7)
    zf = days_f + 719468.0
    era = _fdiv(zf, 146097)
    doe_f = zf - era.astype(jnp.float32) * 146097.0
    yoe = _fdiv(doe_f - _fdiv(doe_f, 1460).astype(jnp.float32)
                + _fdiv(doe_f, 36524).astype(jnp.float32)
                - _fdiv(doe_f, 146096).astype(jnp.float32), 365)
    yoe_f = yoe.astype(jnp.float32)
    doy_f = doe_f - (365.0 * yoe_f + _fdiv(yoe_f, 4).astype(jnp.float32)
                     - _fdiv(yoe_f, 100).astype(jnp.float32))
    mp = _fdiv(5.0 * doy_f + 2.0, 153)
    d0 = doy_f.astype(jnp.int32) - _fdiv(153.0 * mp.astype(jnp.float32) + 2.0, 5)
    m0 = mp + jnp.where(mp < 10, 2, -10)
    return week * 372 + m0 * 31 + d0


def _sc_lookup(t, fused):
    info = plsc.get_sparse_core_info()
    nc, ns = info.num_cores, info.num_subcores
    nw = nc * ns
    bpw = B // nw
    n_chunk = bpw // 128
    mesh = plsc.VectorSubcoreMesh(core_axis_name="c", subcore_axis_name="s")

    stripe = _ROWS_PAD // ns

    @functools.partial(
        pl.kernel, mesh=mesh,
        out_type=jax.ShapeDtypeStruct((B, DIM), jnp.float32),
        scratch_types=[
            pltpu.VMEM((bpw,), jnp.int32),
            pltpu.VMEM((n_chunk, 128), jnp.int32),
            pltpu.VMEM((bpw, DIM), jnp.float32),
            pltpu.VMEM_SHARED((_ROWS_PAD, DIM), jnp.float32),
            pltpu.SemaphoreType.DMA,
            pltpu.SemaphoreType.DMA,
        ],
    )
    def k(t_hbm, fused_hbm, out_hbm, t_v, idx_v, rows_v, f_sp, sem_g, sem_w):
        sid = lax.axis_index("s")
        wid = sid * nc + lax.axis_index("c")
        base = wid * bpw
        stage = pltpu.async_copy(fused_hbm.at[pl.ds(sid * stripe, stripe)],
                                 f_sp.at[pl.ds(sid * stripe, stripe)], sem_g)
        pltpu.sync_copy(t_hbm.at[pl.ds(base, bpw)], t_v)
        for r in range(n_chunk):
            def step(c, carry, r=r):
                tt = t_v[pl.ds(r * 128 + c * 16, 16)]
                idx_v[r, pl.ds(c * 16, 16)] = _date_indices(tt)
                return carry
            lax.fori_loop(0, 128 // 16, step, 0)
        stage.wait()
        plsc.subcore_barrier()
        gathers = [pltpu.async_copy(f_sp.at[idx_v.at[r]],
                                    rows_v.at[pl.ds(r * 128, 128)], sem_g)
                   for r in range(n_chunk)]
        writes = []
        for r in range(n_chunk):
            gathers[r].wait()
            writes.append(pltpu.async_copy(
                rows_v.at[pl.ds(r * 128, 128)],
                out_hbm.at[pl.ds(base + r * 128, 128)], sem_w))
        for wr in writes:
            wr.wait()

    return k(t, fused)


def kernel(t, week_emb, month_emb, day_emb):
    fused = _fuse_tables(week_emb, month_emb, day_emb)
    return _sc_lookup(t.astype(jnp.int32), fused)


# speedup vs baseline: 6.6572x; 1.0002x over previous
import functools

import jax
import jax.numpy as jnp
import numpy as np
from jax import lax
from jax.experimental import pallas as pl
from jax.experimental.pallas import tpu as pltpu
from jax.experimental.pallas import tpu_sc as plsc

B = 16384
DIM = 128
_ROWS = 7 * 12 * 31
_ROWS_PAD = 2688


def _fuse_tables(week_emb, month_emb, day_emb):
    def body(w_ref, m_ref, d_ref, o_ref):
        r = lax.broadcasted_iota(jnp.int32, (_ROWS_PAD, 1), 0)
        rf = r.astype(jnp.float32)
        q31 = _fdiv(rf, 31)
        w = _fdiv(rf, 372)
        m = q31 - 12 * _fdiv(q31.astype(jnp.float32), 12)
        d = r - 31 * q31
        j = lax.broadcasted_iota(jnp.int32, (_ROWS_PAD, 64), 1)
        valid = r < _ROWS
        one = jnp.float32(1.0)
        h = (jnp.where(valid & (j == w), one, 0.0)
             + jnp.where(valid & (j == 7 + m), one, 0.0)
             + jnp.where(valid & (j == 19 + d), one, 0.0))
        t = jnp.concatenate([w_ref[...], m_ref[...], d_ref[...],
                             jnp.zeros((14, DIM), jnp.float32)], axis=0)
        o_ref[...] = jnp.dot(h, t, preferred_element_type=jnp.float32)

    return pl.pallas_call(
        body,
        out_shape=jax.ShapeDtypeStruct((_ROWS_PAD, DIM), jnp.float32),
    )(week_emb, month_emb, day_emb)


def _fdiv(xf, c):
    return ((xf + 0.5) * (1.0 / c)).astype(jnp.int32)


def _date_indices(tt):
    tf = tt.astype(jnp.float32)
    q = (tf * (1.0 / 86400.0)).astype(jnp.int32)
    r = tt - q * 86400
    q = jnp.where(r >= 86400, q + 1, q)
    q = jnp.where(r < 0, q - 1, q)
    days_f = q.astype(jnp.float32)
    week = (q + 3) - 7 * _fdiv(days_f + 3.0, 7)
    zf = days_f + 719468.0
    era = _fdiv(zf, 146097)
    doe_f = zf - era.astype(jnp.float32) * 146097.0
    yoe = _fdiv(doe_f - _fdiv(doe_f, 1460).astype(jnp.float32)
                + _fdiv(doe_f, 36524).astype(jnp.float32)
                - _fdiv(doe_f, 146096).astype(jnp.float32), 365)
    yoe_f = yoe.astype(jnp.float32)
    doy_f = doe_f - (365.0 * yoe_f + _fdiv(yoe_f, 4).astype(jnp.float32)
                     - _fdiv(yoe_f, 100).astype(jnp.float32))
    mp = _fdiv(5.0 * doy_f + 2.0, 153)
    d0 = doy_f.astype(jnp.int32) - _fdiv(153.0 * mp.astype(jnp.float32) + 2.0, 5)
    m0 = mp + jnp.where(mp < 10, 2, -10)
    return week * 372 + m0 * 31 + d0


def _sc_lookup(t, fused):
    info = plsc.get_sparse_core_info()
    nc, ns = info.num_cores, info.num_subcores
    nw = nc * ns
    bpw = B // nw
    n_chunk = bpw // 128
    mesh = plsc.VectorSubcoreMesh(core_axis_name="c", subcore_axis_name="s")

    stripe = _ROWS_PAD // ns

    @functools.partial(
        pl.kernel, mesh=mesh,
        out_type=jax.ShapeDtypeStruct((B, DIM), jnp.float32),
        scratch_types=[
            pltpu.VMEM((bpw,), jnp.int32),
            pltpu.VMEM((n_chunk, 128), jnp.int32),
            pltpu.VMEM((bpw, DIM), jnp.float32),
            pltpu.VMEM_SHARED((_ROWS_PAD, DIM), jnp.float32),
            pltpu.SemaphoreType.DMA,
            pltpu.SemaphoreType.DMA,
        ],
    )
    def k(t_hbm, fused_hbm, out_hbm, t_v, idx_v, rows_v, f_sp, sem_g, sem_w):
        sid = lax.axis_index("s")
        wid = sid * nc + lax.axis_index("c")
        base = wid * bpw
        stage = pltpu.async_copy(fused_hbm.at[pl.ds(sid * stripe, stripe)],
                                 f_sp.at[pl.ds(sid * stripe, stripe)], sem_g)
        pltpu.sync_copy(t_hbm.at[pl.ds(base, bpw)], t_v)
        for r in range(n_chunk):
            def step(c, carry, r=r):
                tt = t_v[pl.ds(r * 128 + c * 16, 16)]
                idx_v[r, pl.ds(c * 16, 16)] = _date_indices(tt)
                return carry
            lax.fori_loop(0, 128 // 16, step, 0)
        stage.wait()
        plsc.subcore_barrier()
        gathers = [pltpu.async_copy(f_sp.at[idx_v.at[r]],
                                    rows_v.at[pl.ds(r * 128, 128)], sem_g)
                   for r in range(n_chunk)]
        writes = []
        for r in range(n_chunk):
            gathers[r].wait()
            writes.append(pltpu.async_copy(
                rows_v.at[pl.ds(r * 128, 128)],
                out_hbm.at[pl.ds(base + r * 128, 128)], sem_w))
        for wr in writes:
            wr.wait()

    return k(t, fused)


def kernel(t, week_emb, month_emb, day_emb):
    fused = _fuse_tables(week_emb, month_emb, day_emb)
    return _sc_lookup(t.astype(jnp.int32), fused)
